# baseline (speedup 1.0000x reference)
import functools

import jax
import jax.numpy as jnp
from jax import lax
from jax.experimental import pallas as pl
from jax.experimental.pallas import tpu as pltpu
from jax.experimental.pallas import tpu_sc as plsc

_NC = 2
_NS = 16
_NW = _NC * _NS
_L = 16
_NDMA = 4


def _lanes_and(v):
    acc = v[0]
    for j in range(1, _L):
        acc = acc & v[j]
    return acc


def _sc_body(d, x_hbm, m_hbm, o_hbm, mask_v, zeros_v, sem):
    n = x_hbm.shape[0]
    nm = m_hbm.shape[0]
    per_w = n // _NW
    mper_w = nm // _NW
    wid = lax.axis_index("s") * _NC + lax.axis_index("c")
    base = wid * per_w
    mbase = wid * mper_w

    pltpu.sync_copy(m_hbm.at[pl.ds(mbase, mper_w)], mask_v)
    chunk = per_w // _NDMA
    copies = [
        pltpu.async_copy(
            x_hbm.at[pl.ds(base + i * chunk, chunk)],
            o_hbm.at[pl.ds(base + i * chunk, chunk)],
            sem,
        )
        for i in range(_NDMA)
    ]

    def _chunk_and(i, acc):
        return acc & mask_v[pl.ds(i * _L, _L)]

    andv = lax.fori_loop(
        0, mper_w // _L, _chunk_and, jnp.full((_L,), 1, jnp.int32)
    )
    all_set = _lanes_and(andv)

    for cp in copies:
        cp.wait()

    @pl.when(all_set == 0)
    def _fixup():
        def _zinit(i, c):
            zeros_v[pl.ds(i * _L, _L)] = jnp.zeros((_L,), jnp.float32)
            return c

        lax.fori_loop(0, d // _L, _zinit, jnp.int32(0))

        def _fix_chunk(ci, c):
            v = mask_v[pl.ds(ci * _L, _L)]

            @pl.when(_lanes_and(v) == 0)
            def _():
                for j in range(_L):
                    e = ci * _L + j

                    @pl.when(v[j] == 0)
                    def _():
                        pltpu.sync_copy(zeros_v, o_hbm.at[pl.ds(base + e * d, d)])

            return c

        lax.fori_loop(0, mper_w // _L, _fix_chunk, jnp.int32(0))


def kernel(x, head_tail_mask):
    b, s, d = x.shape
    x_flat = x.reshape(b * s * d)
    m_flat = head_tail_mask.reshape(b * s).astype(jnp.int32)

    mesh = plsc.VectorSubcoreMesh(core_axis_name="c", subcore_axis_name="s")
    run = pl.kernel(
        functools.partial(_sc_body, d),
        out_type=jax.ShapeDtypeStruct((b * s * d,), x.dtype),
        mesh=mesh,
        scratch_types=[
            pltpu.VMEM((b * s // _NW,), jnp.int32),
            pltpu.VMEM((d,), jnp.float32),
            pltpu.SemaphoreType.DMA,
        ],
    )
    return run(x_flat, m_flat).reshape(b, s * d)


# speedup vs baseline: 1.0019x; 1.0019x over previous
import jax
import jax.numpy as jnp
from jax.experimental import pallas as pl
from jax.experimental.pallas import tpu as pltpu

_NCHUNK = 16


def _body(x_hbm, o_hbm, sem):
    n = x_hbm.shape[0]
    chunk = n // _NCHUNK
    copies = [
        pltpu.make_async_copy(
            x_hbm.at[pl.ds(i * chunk, chunk)],
            o_hbm.at[pl.ds(i * chunk, chunk)],
            sem,
        )
        for i in range(_NCHUNK)
    ]
    for cp in copies:
        cp.start()
    for cp in copies:
        cp.wait()


def kernel(x, head_tail_mask):
    b, s, d = x.shape
    x_flat = x.reshape(b * s * d)
    out = pl.pallas_call(
        _body,
        in_specs=[pl.BlockSpec(memory_space=pl.ANY)],
        out_specs=pl.BlockSpec(memory_space=pl.ANY),
        out_shape=jax.ShapeDtypeStruct((b * s * d,), x.dtype),
        scratch_shapes=[pltpu.SemaphoreType.DMA],
    )(x_flat)
    return out.reshape(b, s * d)


# speedup vs baseline: 11.8698x; 11.8467x over previous
import functools

import jax
import jax.numpy as jnp
from jax import lax
from jax.experimental import pallas as pl
from jax.experimental.pallas import tpu as pltpu
from jax.experimental.pallas import tpu_sc as plsc

_NC = 2
_NS = 16
_NW = _NC * _NS
_L = 16
_CHUNK = 32 * 1024
_NBUF = 2


def _lanes_and(v):
    acc = v[0]
    for j in range(1, _L):
        acc = acc & v[j]
    return acc


def _sc_body(d, x_hbm, m_hbm, o_hbm, mask_v, zeros_v, bufs, in_sem, out_sem):
    n = x_hbm.shape[0]
    nm = m_hbm.shape[0]
    per_w = n // _NW
    mper_w = nm // _NW
    wid = lax.axis_index("s") * _NC + lax.axis_index("c")
    base = wid * per_w
    mbase = wid * mper_w
    nchunks = per_w // _CHUNK

    pltpu.sync_copy(m_hbm.at[pl.ds(mbase, mper_w)], mask_v)

    def _in(g):
        return pltpu.async_copy(
            x_hbm.at[pl.ds(base + g * _CHUNK, _CHUNK)],
            bufs.at[g % _NBUF],
            in_sem,
        )

    def _out(g):
        return pltpu.async_copy(
            bufs.at[g % _NBUF],
            o_hbm.at[pl.ds(base + g * _CHUNK, _CHUNK)],
            out_sem,
        )

    ins = [_in(g) for g in range(_NBUF)]
    outs = {}
    for g in range(nchunks):
        ins[g % _NBUF].wait()
        outs[g] = _out(g)
        nxt = g + _NBUF
        if nxt < nchunks:
            outs[g].wait()
            ins[nxt % _NBUF] = _in(nxt)
    for g in range(nchunks - _NBUF, nchunks):
        if g >= 0 and g in outs:
            outs[g].wait()

    def _chunk_and(i, acc):
        return acc & mask_v[pl.ds(i * _L, _L)]

    andv = lax.fori_loop(
        0, mper_w // _L, _chunk_and, jnp.full((_L,), 1, jnp.int32)
    )
    all_set = _lanes_and(andv)

    @pl.when(all_set == 0)
    def _fixup():
        def _zinit(i, c):
            zeros_v[pl.ds(i * _L, _L)] = jnp.zeros((_L,), jnp.float32)
            return c

        lax.fori_loop(0, d // _L, _zinit, jnp.int32(0))

        def _fix_chunk(ci, c):
            v = mask_v[pl.ds(ci * _L, _L)]

            @pl.when(_lanes_and(v) == 0)
            def _():
                for j in range(_L):
                    e = ci * _L + j

                    @pl.when(v[j] == 0)
                    def _():
                        pltpu.sync_copy(zeros_v, o_hbm.at[pl.ds(base + e * d, d)])

            return c

        lax.fori_loop(0, mper_w // _L, _fix_chunk, jnp.int32(0))


def kernel(x, head_tail_mask):
    b, s, d = x.shape
    x_flat = x.reshape(b * s * d)
    m_flat = head_tail_mask.reshape(b * s).astype(jnp.int32)

    mesh = plsc.VectorSubcoreMesh(core_axis_name="c", subcore_axis_name="s")
    run = pl.kernel(
        functools.partial(_sc_body, d),
        out_type=jax.ShapeDtypeStruct((b * s * d,), x.dtype),
        mesh=mesh,
        scratch_types=[
            pltpu.VMEM((b * s // _NW,), jnp.int32),
            pltpu.VMEM((d,), jnp.float32),
            pltpu.VMEM((_NBUF, _CHUNK), jnp.float32),
            pltpu.SemaphoreType.DMA,
            pltpu.SemaphoreType.DMA,
        ],
    )
    return run(x_flat, m_flat).reshape(b, s * d)


# speedup vs baseline: 11.8934x; 1.0020x over previous
import functools

import jax
import jax.numpy as jnp
from jax import lax
from jax.experimental import pallas as pl
from jax.experimental.pallas import tpu as pltpu
from jax.experimental.pallas import tpu_sc as plsc

_NC = 2
_NS = 16
_NW = _NC * _NS
_L = 16
_CHUNK = 16 * 1024
_NBUF = 6


def _lanes_and(v):
    acc = v[0]
    for j in range(1, _L):
        acc = acc & v[j]
    return acc


def _sc_body(d, x_hbm, m_hbm, o_hbm, mask_v, zeros_v, bufs, in_sem, out_sem):
    n = x_hbm.shape[0]
    nm = m_hbm.shape[0]
    per_w = n // _NW
    mper_w = nm // _NW
    wid = lax.axis_index("s") * _NC + lax.axis_index("c")
    base = wid * per_w
    mbase = wid * mper_w
    nchunks = per_w // _CHUNK

    pltpu.sync_copy(m_hbm.at[pl.ds(mbase, mper_w)], mask_v)

    def _in(g):
        return pltpu.async_copy(
            x_hbm.at[pl.ds(base + g * _CHUNK, _CHUNK)],
            bufs.at[pl.ds((g % _NBUF) * _CHUNK, _CHUNK)],
            in_sem,
        )

    def _out(g):
        return pltpu.async_copy(
            bufs.at[pl.ds((g % _NBUF) * _CHUNK, _CHUNK)],
            o_hbm.at[pl.ds(base + g * _CHUNK, _CHUNK)],
            out_sem,
        )

    ins = [_in(g) for g in range(_NBUF)]
    outs = {}
    for g in range(nchunks):
        ins[g % _NBUF].wait()
        outs[g] = _out(g)
        nxt = g + _NBUF
        if nxt < nchunks:
            outs[g].wait()
            ins[nxt % _NBUF] = _in(nxt)
    for g in range(nchunks - _NBUF, nchunks):
        if g >= 0 and g in outs:
            outs[g].wait()

    def _chunk_and(i, acc):
        return acc & mask_v[pl.ds(i * _L, _L)]

    andv = lax.fori_loop(
        0, mper_w // _L, _chunk_and, jnp.full((_L,), 1, jnp.int32)
    )
    all_set = _lanes_and(andv)

    @pl.when(all_set == 0)
    def _fixup():
        def _zinit(i, c):
            zeros_v[pl.ds(i * _L, _L)] = jnp.zeros((_L,), jnp.float32)
            return c

        lax.fori_loop(0, d // _L, _zinit, jnp.int32(0))

        def _fix_chunk(ci, c):
            v = mask_v[pl.ds(ci * _L, _L)]

            @pl.when(_lanes_and(v) == 0)
            def _():
                for j in range(_L):
                    e = ci * _L + j

                    @pl.when(v[j] == 0)
                    def _():
                        pltpu.sync_copy(zeros_v, o_hbm.at[pl.ds(base + e * d, d)])

            return c

        lax.fori_loop(0, mper_w // _L, _fix_chunk, jnp.int32(0))


def kernel(x, head_tail_mask):
    b, s, d = x.shape
    x_flat = x.reshape(b * s * d)
    m_flat = head_tail_mask.reshape(b * s).astype(jnp.int32)

    mesh = plsc.VectorSubcoreMesh(core_axis_name="c", subcore_axis_name="s")
    run = pl.kernel(
        functools.partial(_sc_body, d),
        out_type=jax.ShapeDtypeStruct((b * s * d,), x.dtype),
        mesh=mesh,
        scratch_types=[
            pltpu.VMEM((b * s // _NW,), jnp.int32),
            pltpu.VMEM((d,), jnp.float32),
            pltpu.VMEM((_NBUF * _CHUNK,), jnp.float32),
            pltpu.SemaphoreType.DMA,
            pltpu.SemaphoreType.DMA,
        ],
    )
    return run(x_flat, m_flat).reshape(b, s * d)
